# Initial kernel scaffold; baseline (speedup 1.0000x reference)
import jax
import jax.numpy as jnp
from jax.experimental import pallas as pl


def kernel(x, edge_index, beta):
    raise NotImplementedError("write your pallas kernel here")


# speedup vs baseline: 1.9371x; 1.9371x over previous
import dataclasses
import functools

import jax
import jax.numpy as jnp
from jax.experimental import pallas as pl
from jax.experimental.pallas import tpu as pltpu
from jax.experimental.pallas import tpu_sc as plsc

N = 10000
E = 160000
D = 256
H = D // 2
NC = 2
NS = 16
L = 16
CHUNK = 64
NCHUNKS = E // CHUNK
RCHUNK = 200
NRCH = N // RCHUNK

_mesh = plsc.VectorSubcoreMesh(
    core_axis_name="c", subcore_axis_name="s", num_cores=NC, num_subcores=NS
)

_sc_params = pltpu.CompilerParams()
if "needs_layout_passes" in pltpu.CompilerParams.__dataclass_fields__:
    _sc_params = dataclasses.replace(_sc_params, needs_layout_passes=False)


def _prep_body(x_ref, xa_ref, xb_ref, rn_ref):
    xx = x_ref[...]
    xa_ref[...] = xx[:, :H]
    xb_ref[...] = xx[:, H:]
    ss = jnp.sum(xx * xx, axis=1, keepdims=True)
    rn_ref[...] = 1.0 / jnp.maximum(jnp.sqrt(ss), 1e-12)


_prep = pl.pallas_call(
    _prep_body,
    out_shape=[
        jax.ShapeDtypeStruct((N, H), jnp.float32),
        jax.ShapeDtypeStruct((N, H), jnp.float32),
        jax.ShapeDtypeStruct((N, 1), jnp.float32),
    ],
)


def _score_body(x_hbm, row_hbm, col_hbm, rn_hbm, beta_hbm, zeros_hbm,
                w_hbm, se_hbm,
                idxr_v, idxc_v, xr_v, xc_v, w_v, rn_v, beta_v, se_tmp_v,
                se_sh, sem1, sem2):
    c = jax.lax.axis_index("c")
    s = jax.lax.axis_index("s")
    wid = c * NS + s
    pltpu.sync_copy(rn_hbm, rn_v)
    pltpu.sync_copy(beta_hbm, beta_v)

    @pl.when(s == 0)
    def _():
        pltpu.sync_copy(zeros_hbm, se_sh)

    plsc.subcore_barrier()

    beta_vec = beta_v[...]
    lanes = jax.lax.iota(jnp.int32, L)
    nch = (NCHUNKS - wid + 31) // 32

    @pl.loop(0, nch)
    def _chunk(j):
        base = (wid + j * 32) * CHUNK
        pltpu.sync_copy(row_hbm.at[pl.ds(base, CHUNK)], idxr_v)
        pltpu.sync_copy(col_hbm.at[pl.ds(base, CHUNK)], idxc_v)
        d1 = pltpu.async_copy(x_hbm.at[idxr_v], xr_v, sem1)
        d2 = pltpu.async_copy(x_hbm.at[idxc_v], xc_v, sem2)
        d1.wait()
        d2.wait()
        for g in range(CHUNK // L):
            rowsel = lanes + g * L

            def dbody(d, acc):
                dv = jnp.broadcast_to(d, (L,))
                return acc + (plsc.load_gather(xr_v, [rowsel, dv])
                              * plsc.load_gather(xc_v, [rowsel, dv]))

            dots = jax.lax.fori_loop(0, D, dbody, jnp.zeros((L,), jnp.float32),
                                     unroll=4)
            rr = plsc.load_gather(rn_v, [idxr_v[pl.ds(g * L, L)]])
            rc = plsc.load_gather(rn_v, [idxc_v[pl.ds(g * L, L)]])
            w_v[pl.ds(g * L, L)] = jnp.exp(dots * rr * rc * beta_vec)
        pltpu.sync_copy(w_v, w_hbm.at[pl.ds(base, CHUNK)])
        pltpu.sync_copy(w_v, se_sh.at[idxc_v], add=True)

    plsc.subcore_barrier()

    @pl.when(s == 0)
    def _():
        pltpu.sync_copy(se_sh, se_tmp_v)
        pltpu.sync_copy(se_tmp_v, se_hbm.at[pl.ds(c * N, N)])


_score = functools.partial(
    pl.kernel,
    out_type=[
        jax.ShapeDtypeStruct((E,), jnp.float32),
        jax.ShapeDtypeStruct((NC * N,), jnp.float32),
    ],
    mesh=_mesh,
    compiler_params=_sc_params,
    scratch_types=[
        pltpu.VMEM((CHUNK,), jnp.int32),
        pltpu.VMEM((CHUNK,), jnp.int32),
        pltpu.VMEM((CHUNK, D), jnp.float32),
        pltpu.VMEM((CHUNK, D), jnp.float32),
        pltpu.VMEM((CHUNK,), jnp.float32),
        pltpu.VMEM((N,), jnp.float32),
        pltpu.VMEM((L,), jnp.float32),
        pltpu.VMEM((N,), jnp.float32),
        pltpu.VMEM_SHARED((N,), jnp.float32),
        pltpu.SemaphoreType.DMA,
        pltpu.SemaphoreType.DMA,
    ],
)(_score_body)


def _scatter_body(xa_hbm, xb_hbm, row_hbm, col_hbm, w_hbm, zacc_hbm,
                  acc_hbm,
                  idxr_v, idxc_v, w_v, rows_v, obuf_v, acc_sh, sem):
    c = jax.lax.axis_index("c")
    s = jax.lax.axis_index("s")

    @pl.loop(0, (NRCH - s + NS - 1) // NS)
    def _zero(t):
        r0 = (s + t * NS) * RCHUNK
        pltpu.sync_copy(zacc_hbm.at[pl.ds(r0, RCHUNK)],
                        acc_sh.at[pl.ds(r0, RCHUNK)])

    plsc.subcore_barrier()

    nch = (NCHUNKS - s + 15) // 16

    @pl.loop(0, nch)
    def _chunk(j):
        base = (s + j * 16) * CHUNK
        pltpu.sync_copy(row_hbm.at[pl.ds(base, CHUNK)], idxr_v)
        pltpu.sync_copy(col_hbm.at[pl.ds(base, CHUNK)], idxc_v)
        pltpu.sync_copy(w_hbm.at[pl.ds(base, CHUNK)], w_v)

        @pl.when(c == 0)
        def _():
            pltpu.async_copy(xa_hbm.at[idxr_v], rows_v, sem).wait()

        @pl.when(c == 1)
        def _():
            pltpu.async_copy(xb_hbm.at[idxr_v], rows_v, sem).wait()

        @pl.loop(0, CHUNK)
        def _edge(e):
            ws = plsc.load_gather(w_v, [jnp.broadcast_to(e, (L,))])
            for k in range(H // L):
                sl = (e, pl.ds(k * L, L))
                rows_v[sl] = rows_v[sl] * ws

        pltpu.sync_copy(rows_v, acc_sh.at[idxc_v], add=True)

    plsc.subcore_barrier()

    @pl.loop(0, (NRCH - s + NS - 1) // NS)
    def _drain(t):
        r0 = (s + t * NS) * RCHUNK
        pltpu.sync_copy(acc_sh.at[pl.ds(r0, RCHUNK)], obuf_v)
        pltpu.sync_copy(obuf_v, acc_hbm.at[c].at[pl.ds(r0, RCHUNK)])


_scatter = functools.partial(
    pl.kernel,
    out_type=jax.ShapeDtypeStruct((NC, N, H), jnp.float32),
    mesh=_mesh,
    compiler_params=_sc_params,
    scratch_types=[
        pltpu.VMEM((CHUNK,), jnp.int32),
        pltpu.VMEM((CHUNK,), jnp.int32),
        pltpu.VMEM((CHUNK,), jnp.float32),
        pltpu.VMEM((CHUNK, H), jnp.float32),
        pltpu.VMEM((RCHUNK, H), jnp.float32),
        pltpu.VMEM_SHARED((N, H), jnp.float32),
        pltpu.SemaphoreType.DMA,
    ],
)(_scatter_body)


def _final_body(acca_ref, accb_ref, s0_ref, s1_ref, out_ref):
    inv = 1.0 / jnp.maximum(s0_ref[...] + s1_ref[...], 1e-16)
    out_ref[:, :H] = acca_ref[...] * inv
    out_ref[:, H:] = accb_ref[...] * inv


_final = pl.pallas_call(
    _final_body,
    out_shape=jax.ShapeDtypeStruct((N, D), jnp.float32),
)


def kernel(x, edge_index, beta):
    x = x.astype(jnp.float32)
    row = edge_index[0].astype(jnp.int32)
    col = edge_index[1].astype(jnp.int32)
    beta16 = jnp.broadcast_to(beta.astype(jnp.float32), (L,))
    zeros_n = jnp.zeros((N,), jnp.float32)
    zacc = jnp.zeros((N, H), jnp.float32)

    xa, xb, rn2 = _prep(x)
    rn = rn2.reshape(N)
    w, sumexp = _score(x, row, col, rn, beta16, zeros_n)
    acc = _scatter(xa, xb, row, col, w, zacc)
    out = _final(acc[0], acc[1],
                 sumexp[:N].reshape(N, 1), sumexp[N:].reshape(N, 1))
    return out


# speedup vs baseline: 6.6827x; 3.4498x over previous
import dataclasses
import functools

import jax
import jax.numpy as jnp
from jax.experimental import pallas as pl
from jax.experimental.pallas import tpu as pltpu
from jax.experimental.pallas import tpu_sc as plsc

N = 10000
E = 160000
D = 256
H = D // 2
NC = 2
NS = 16
L = 16
CHUNK = 64
NCHUNKS = E // CHUNK
RCHUNK = 128
NRCH = N // RCHUNK
RTAIL = N - NRCH * RCHUNK

_mesh = plsc.VectorSubcoreMesh(
    core_axis_name="c", subcore_axis_name="s", num_cores=NC, num_subcores=NS
)

_sc_params = pltpu.CompilerParams()
if "needs_layout_passes" in pltpu.CompilerParams.__dataclass_fields__:
    _sc_params = dataclasses.replace(_sc_params, needs_layout_passes=False)


def _prep_body(x_ref, xa_ref, xb_ref, rn_ref):
    xx = x_ref[...]
    xa_ref[...] = xx[:, :H]
    xb_ref[...] = xx[:, H:]
    ss = jnp.sum(xx * xx, axis=1, keepdims=True)
    rn_ref[...] = 1.0 / jnp.maximum(jnp.sqrt(ss), 1e-12)


_prep = pl.pallas_call(
    _prep_body,
    out_shape=[
        jax.ShapeDtypeStruct((N, H), jnp.float32),
        jax.ShapeDtypeStruct((N, H), jnp.float32),
        jax.ShapeDtypeStruct((N, 1), jnp.float32),
    ],
)


def _score_body(x_hbm, row_hbm, col_hbm, rn_hbm, beta_hbm, zeros_hbm,
                w_hbm, se_hbm,
                idxr0, idxc0, xr0, xc0, semr0, semc0,
                idxr1, idxc1, xr1, xc1, semr1, semc1,
                w_v, rn_v, beta_v, se_tmp_v, se_sh):
    c = jax.lax.axis_index("c")
    s = jax.lax.axis_index("s")
    wid = c * NS + s
    pltpu.sync_copy(rn_hbm, rn_v)
    pltpu.sync_copy(beta_hbm, beta_v)

    @pl.when(s == 0)
    def _():
        pltpu.sync_copy(zeros_hbm, se_sh)

    plsc.subcore_barrier()

    beta_vec = beta_v[...]
    lanes = jax.lax.iota(jnp.int32, L)
    last_lane = lanes == (L - 1)
    nch = (NCHUNKS - wid + 31) // 32

    def chunk_base(q):
        return (wid + q * 32) * CHUNK

    def prefetch(q, idxr_v, idxc_v, xr_v, xc_v, semr, semc):
        base = chunk_base(q)
        pltpu.sync_copy(row_hbm.at[pl.ds(base, CHUNK)], idxr_v)
        pltpu.sync_copy(col_hbm.at[pl.ds(base, CHUNK)], idxc_v)
        pltpu.async_copy(x_hbm.at[idxr_v], xr_v, semr)
        pltpu.async_copy(x_hbm.at[idxc_v], xc_v, semc)

    def compute(j, idxr_v, idxc_v, xr_v, xc_v, semr, semc):
        base = chunk_base(j)
        pltpu.make_async_copy(x_hbm.at[idxr_v], xr_v, semr).wait()
        pltpu.make_async_copy(x_hbm.at[idxc_v], xc_v, semc).wait()

        @pl.loop(0, CHUNK, unroll=4)
        def _edge(e):
            acc = xr_v[e, pl.ds(0, L)] * xc_v[e, pl.ds(0, L)]
            for k in range(1, D // L):
                acc += xr_v[e, pl.ds(k * L, L)] * xc_v[e, pl.ds(k * L, L)]
            sc = plsc.cumsum(acc)
            plsc.store_scatter(w_v, [jnp.broadcast_to(e, (L,))], sc,
                               mask=last_lane)

        for g in range(CHUNK // L):
            rr = plsc.load_gather(rn_v, [idxr_v[pl.ds(g * L, L)]])
            rc = plsc.load_gather(rn_v, [idxc_v[pl.ds(g * L, L)]])
            dots = w_v[pl.ds(g * L, L)]
            w_v[pl.ds(g * L, L)] = jnp.exp(dots * rr * rc * beta_vec)
        pltpu.sync_copy(w_v, w_hbm.at[pl.ds(base, CHUNK)])
        pltpu.sync_copy(w_v, se_sh.at[idxc_v], add=True)

    prefetch(0, idxr0, idxc0, xr0, xc0, semr0, semc0)

    @pl.loop(0, nch)
    def _chunk(j):
        @pl.when(j % 2 == 0)
        def _():
            @pl.when(j + 1 < nch)
            def _():
                prefetch(j + 1, idxr1, idxc1, xr1, xc1, semr1, semc1)
            compute(j, idxr0, idxc0, xr0, xc0, semr0, semc0)

        @pl.when(j % 2 == 1)
        def _():
            @pl.when(j + 1 < nch)
            def _():
                prefetch(j + 1, idxr0, idxc0, xr0, xc0, semr0, semc0)
            compute(j, idxr1, idxc1, xr1, xc1, semr1, semc1)

    plsc.subcore_barrier()

    @pl.when(s == 0)
    def _():
        pltpu.sync_copy(se_sh, se_tmp_v)
        pltpu.sync_copy(se_tmp_v, se_hbm.at[pl.ds(c * N, N)])


_score = functools.partial(
    pl.kernel,
    out_type=[
        jax.ShapeDtypeStruct((E,), jnp.float32),
        jax.ShapeDtypeStruct((NC * N,), jnp.float32),
    ],
    mesh=_mesh,
    compiler_params=_sc_params,
    scratch_types=[
        pltpu.VMEM((CHUNK,), jnp.int32),
        pltpu.VMEM((CHUNK,), jnp.int32),
        pltpu.VMEM((CHUNK, D), jnp.float32),
        pltpu.VMEM((CHUNK, D), jnp.float32),
        pltpu.SemaphoreType.DMA,
        pltpu.SemaphoreType.DMA,
        pltpu.VMEM((CHUNK,), jnp.int32),
        pltpu.VMEM((CHUNK,), jnp.int32),
        pltpu.VMEM((CHUNK, D), jnp.float32),
        pltpu.VMEM((CHUNK, D), jnp.float32),
        pltpu.SemaphoreType.DMA,
        pltpu.SemaphoreType.DMA,
        pltpu.VMEM((CHUNK,), jnp.float32),
        pltpu.VMEM((N,), jnp.float32),
        pltpu.VMEM((L,), jnp.float32),
        pltpu.VMEM((N,), jnp.float32),
        pltpu.VMEM_SHARED((N,), jnp.float32),
    ],
)(_score_body)


CCHUNK = 128
NCCH = E // CCHUNK


def _scatter_body(xa_hbm, xb_hbm, row_hbm, col_hbm, w_hbm, zacc_hbm,
                  acc_hbm,
                  idxr0, idxc0, w0, rows0, semg0, sems0,
                  idxr1, idxc1, w1, rows1, semg1, sems1,
                  acc_sh):
    c = jax.lax.axis_index("c")
    s = jax.lax.axis_index("s")

    @pl.loop(0, (NRCH - s + NS - 1) // NS)
    def _zero(t):
        r0 = (s + t * NS) * RCHUNK
        pltpu.sync_copy(zacc_hbm.at[pl.ds(r0, RCHUNK)],
                        acc_sh.at[pl.ds(r0, RCHUNK)])

    @pl.when(s == 0)
    def _():
        pltpu.sync_copy(zacc_hbm.at[pl.ds(NRCH * RCHUNK, RTAIL)],
                        acc_sh.at[pl.ds(NRCH * RCHUNK, RTAIL)])

    plsc.subcore_barrier()

    nch = (NCCH - s + 15) // 16

    def chunk_base(q):
        return (s + q * 16) * CCHUNK

    def prefetch(q, idxr_v, idxc_v, w_v, rows_v, semg, sems):
        @pl.when(q >= 2)
        def _():
            pltpu.make_async_copy(rows_v, acc_sh.at[idxc_v], sems).wait()
        base = chunk_base(q)
        pltpu.sync_copy(row_hbm.at[pl.ds(base, CCHUNK)], idxr_v)
        pltpu.sync_copy(col_hbm.at[pl.ds(base, CCHUNK)], idxc_v)
        pltpu.sync_copy(w_hbm.at[pl.ds(base, CCHUNK)], w_v)

        @pl.when(c == 0)
        def _():
            pltpu.async_copy(xa_hbm.at[idxr_v], rows_v, semg)

        @pl.when(c == 1)
        def _():
            pltpu.async_copy(xb_hbm.at[idxr_v], rows_v, semg)

    def compute(idxr_v, idxc_v, w_v, rows_v, semg, sems):
        @pl.when(c == 0)
        def _():
            pltpu.make_async_copy(xa_hbm.at[idxr_v], rows_v, semg).wait()

        @pl.when(c == 1)
        def _():
            pltpu.make_async_copy(xb_hbm.at[idxr_v], rows_v, semg).wait()

        @pl.loop(0, CCHUNK, unroll=4)
        def _edge(e):
            ws = plsc.load_gather(w_v, [jnp.broadcast_to(e, (L,))])
            for k in range(H // L):
                sl = (e, pl.ds(k * L, L))
                rows_v[sl] = rows_v[sl] * ws

        pltpu.async_copy(rows_v, acc_sh.at[idxc_v], sems, add=True)

    prefetch(0, idxr0, idxc0, w0, rows0, semg0, sems0)

    @pl.loop(0, nch)
    def _chunk(j):
        @pl.when(j % 2 == 0)
        def _():
            @pl.when(j + 1 < nch)
            def _():
                prefetch(j + 1, idxr1, idxc1, w1, rows1, semg1, sems1)
            compute(idxr0, idxc0, w0, rows0, semg0, sems0)

        @pl.when(j % 2 == 1)
        def _():
            @pl.when(j + 1 < nch)
            def _():
                prefetch(j + 1, idxr0, idxc0, w0, rows0, semg0, sems0)
            compute(idxr1, idxc1, w1, rows1, semg1, sems1)

    pltpu.make_async_copy(rows0, acc_sh.at[idxc0], sems0).wait()
    pltpu.make_async_copy(rows1, acc_sh.at[idxc1], sems1).wait()

    plsc.subcore_barrier()

    @pl.loop(0, (NRCH - s + NS - 1) // NS)
    def _drain(t):
        r0 = (s + t * NS) * RCHUNK
        pltpu.sync_copy(acc_sh.at[pl.ds(r0, RCHUNK)], rows0)
        pltpu.sync_copy(rows0, acc_hbm.at[c].at[pl.ds(r0, RCHUNK)])

    @pl.when(s == 0)
    def _():
        r0 = NRCH * RCHUNK
        pltpu.sync_copy(acc_sh.at[pl.ds(r0, RTAIL)], rows1.at[pl.ds(0, RTAIL)])
        pltpu.sync_copy(rows1.at[pl.ds(0, RTAIL)],
                        acc_hbm.at[c].at[pl.ds(r0, RTAIL)])


_scatter = functools.partial(
    pl.kernel,
    out_type=jax.ShapeDtypeStruct((NC, N, H), jnp.float32),
    mesh=_mesh,
    compiler_params=_sc_params,
    scratch_types=[
        pltpu.VMEM((CCHUNK,), jnp.int32),
        pltpu.VMEM((CCHUNK,), jnp.int32),
        pltpu.VMEM((CCHUNK,), jnp.float32),
        pltpu.VMEM((CCHUNK, H), jnp.float32),
        pltpu.SemaphoreType.DMA,
        pltpu.SemaphoreType.DMA,
        pltpu.VMEM((CCHUNK,), jnp.int32),
        pltpu.VMEM((CCHUNK,), jnp.int32),
        pltpu.VMEM((CCHUNK,), jnp.float32),
        pltpu.VMEM((CCHUNK, H), jnp.float32),
        pltpu.SemaphoreType.DMA,
        pltpu.SemaphoreType.DMA,
        pltpu.VMEM_SHARED((N, H), jnp.float32),
    ],
)(_scatter_body)


def _final_body(acca_ref, accb_ref, s0_ref, s1_ref, out_ref):
    inv = 1.0 / jnp.maximum(s0_ref[...] + s1_ref[...], 1e-16)
    out_ref[:, :H] = acca_ref[...] * inv
    out_ref[:, H:] = accb_ref[...] * inv


_final = pl.pallas_call(
    _final_body,
    out_shape=jax.ShapeDtypeStruct((N, D), jnp.float32),
)


def kernel(x, edge_index, beta):
    x = x.astype(jnp.float32)
    row = edge_index[0].astype(jnp.int32)
    col = edge_index[1].astype(jnp.int32)
    beta16 = jnp.broadcast_to(beta.astype(jnp.float32), (L,))
    zeros_n = jnp.zeros((N,), jnp.float32)
    zacc = jnp.zeros((N, H), jnp.float32)

    xa, xb, rn2 = _prep(x)
    rn = rn2.reshape(N)
    w, sumexp = _score(x, row, col, rn, beta16, zeros_n)
    acc = _scatter(xa, xb, row, col, w, zacc)
    out = _final(acc[0], acc[1],
                 sumexp[:N].reshape(N, 1), sumexp[N:].reshape(N, 1))
    return out
